# Initial kernel scaffold; baseline (speedup 1.0000x reference)
import jax
import jax.numpy as jnp
from jax.experimental import pallas as pl


def kernel(mention_features, mention_locations, entity_embeddings, W1, b1, W2, b2, Wq, bq, Wk, bk, Wv, bv, We, be, Wskip, bskip):
    raise NotImplementedError("write your pallas kernel here")


# speedup vs baseline: 794.6944x; 794.6944x over previous
import functools

import jax
import jax.numpy as jnp
from jax.experimental import pallas as pl

N = 512
D = 128
HEADS = 8
C = D // HEADS
LAYERS = 2


def _fused_body(x_ref, w1s_ref, w1d_ref, b1_ref, w2_ref, b2_ref,
                wq_ref, bq_ref, wk_ref, bk_ref, wv_ref, bv_ref,
                we_ref, be_ref, ws_ref, bs_ref, out_ref):
    x = x_ref[...]

    at = jax.lax.dot_general(w1s_ref[...], x,
                             (((0,), (1,)), ((), ())),
                             preferred_element_type=jnp.float32, precision=jax.lax.Precision.HIGHEST)
    b = jnp.dot(x, w1d_ref[...], preferred_element_type=jnp.float32, precision=jax.lax.Precision.HIGHEST)
    b = b + b1_ref[...]

    w2 = w2_ref[...]
    z = jnp.zeros((N, N), jnp.float32)
    for c in range(D):
        arow = at[c:c + 1, :]
        bcol = b[:, c:c + 1]
        z = z + jnp.maximum(bcol + arow, 0.0) * w2[0, c]
    z = z + b2_ref[...]
    ewt = jax.nn.sigmoid(z)
    mask = ewt > 0.5

    inv_sqrt_c = 1.0 / (C ** 0.5)

    for l in range(LAYERS):
        q = jnp.dot(x, wq_ref[l], preferred_element_type=jnp.float32, precision=jax.lax.Precision.HIGHEST) + bq_ref[l]
        k = jnp.dot(x, wk_ref[l], preferred_element_type=jnp.float32, precision=jax.lax.Precision.HIGHEST) + bk_ref[l]
        v = jnp.dot(x, wv_ref[l], preferred_element_type=jnp.float32, precision=jax.lax.Precision.HIGHEST) + bv_ref[l]
        wef = we_ref[l]
        bef = be_ref[l]

        outs = []
        for h in range(HEADS):
            sl = slice(h * C, (h + 1) * C)
            qh, kh, vh = q[:, sl], k[:, sl], v[:, sl]
            weh = wef[:, sl]
            beh = bef[:, sl]

            s = jax.lax.dot_general(qh, kh, (((1,), (1,)), ((), ())),
                                    preferred_element_type=jnp.float32, precision=jax.lax.Precision.HIGHEST)
            qwe = jax.lax.dot_general(qh, weh, (((1,), (1,)), ((), ())),
                                      preferred_element_type=jnp.float32, precision=jax.lax.Precision.HIGHEST)
            qbe = jax.lax.dot_general(qh, beh, (((1,), (1,)), ((), ())),
                                      preferred_element_type=jnp.float32, precision=jax.lax.Precision.HIGHEST)

            logits = (s + ewt * qwe + qbe) * inv_sqrt_c
            m = jnp.max(jnp.where(mask, logits, -1e30), axis=1, keepdims=True)
            m = jnp.where(m < -1e29, 0.0, m)
            ex = jnp.where(mask, jnp.exp(logits - m), 0.0)
            den = jnp.sum(ex, axis=1, keepdims=True)
            r = 1.0 / (den + 1e-16)
            alpha = ex * r
            outv = jnp.dot(alpha, vh, preferred_element_type=jnp.float32, precision=jax.lax.Precision.HIGHEST)
            sew = jnp.sum(alpha * ewt, axis=1, keepdims=True)
            sa = den * r
            outs.append(outv + sew * weh + sa * beh)

        attn = jnp.concatenate(outs, axis=1)
        skip = jnp.dot(x, ws_ref[l], preferred_element_type=jnp.float32, precision=jax.lax.Precision.HIGHEST) + bs_ref[l]
        x = x + attn + skip

    out_ref[...] = x


@functools.partial(jax.jit, static_argnames=())
def kernel(mention_features, mention_locations, entity_embeddings,
           W1, b1, W2, b2, Wq, bq, Wk, bk, Wv, bv, We, be, Wskip, bskip):
    del mention_features, mention_locations
    x = entity_embeddings
    w1s = W1[:D]
    w1d = W1[D:]
    b1r = b1.reshape(1, D)
    w2r = W2.reshape(1, D)
    b2r = b2.reshape(1, 1)
    wer = We.reshape(LAYERS, 1, D)
    ber = be.reshape(LAYERS, 1, D)
    bqr = bq.reshape(LAYERS, 1, D)
    bkr = bk.reshape(LAYERS, 1, D)
    bvr = bv.reshape(LAYERS, 1, D)
    bsr = bskip.reshape(LAYERS, 1, D)

    return pl.pallas_call(
        _fused_body,
        out_shape=jax.ShapeDtypeStruct((N, D), jnp.float32),
    )(x, w1s, w1d, b1r, w2r, b2r, Wq, bqr, Wk, bkr, Wv, bvr, wer, ber, Wskip, bsr)
